# baseline (speedup 1.0000x reference)
import functools

import jax
import jax.numpy as jnp
from jax.experimental import pallas as pl
_K = 3


def _unfold_body(x_ref, d_ref, o_ref, *, cb, w, u_dim, l_dim, su):
    nstrips = u_dim // su
    n_groups = cb * _K * _K // 8
    for st in range(nstrips):
        u0 = st * su
        ds = d_ref[0, u0:u0 + su, :]
        is0 = (ds == 0)
        is1 = (ds == 1)
        fi = ((jax.lax.broadcasted_iota(jnp.int32, (su, l_dim), 0) + u0) * l_dim
              + jax.lax.broadcasted_iota(jnp.int32, (su, l_dim), 1))
        j = fi - (fi // w) * w
        col_ok = {s: (j + s >= 0) & (j + s < w) for s in (-1, 1, 3)}

        xs_cache = {}

        def get_xs(c):
            if c not in xs_cache:
                lo, hi = max(u0 - 2, 0), min(u0 + su + 6, u_dim)
                v = x_ref[0, c, lo:hi, :]
                if u0 - 2 < 0:
                    v = jnp.concatenate(
                        [jnp.zeros((2 - u0, l_dim), v.dtype), v], axis=0)
                if u0 + su + 6 > u_dim:
                    v = jnp.concatenate(
                        [v, jnp.zeros((u0 + su + 6 - u_dim, l_dim), v.dtype)],
                        axis=0)
                xs_cache[c] = v
            return xs_cache[c]

        v0_cache = {}

        def tap_value(c, r, s):
            xs = get_xs(c)
            q, m = divmod(r * w + s, l_dim)

            def sh(qq, mm):
                v = xs[2 + qq:2 + qq + su,
                       max(mm, 0):l_dim + min(mm, 0)]
                if mm > 0:
                    v = jnp.concatenate(
                        [v, jnp.zeros((su, mm), v.dtype)], axis=1)
                elif mm < 0:
                    v = jnp.concatenate(
                        [jnp.zeros((su, -mm), v.dtype), v], axis=1)
                return v

            v = sh(q, m) if m == 0 else sh(q, m) + sh(q + 1, m - l_dim)
            if s in col_ok:
                v = jnp.where(col_ok[s], v, 0.0)
            return v

        for g in range(n_groups):
            rows = []
            for sub in range(8):
                rl = 8 * g + sub
                c, t = rl // 9, rl % 9
                ki, kj = t // _K, t % _K
                if c not in v0_cache:
                    v0_cache[c] = tap_value(c, -1, -1)
                v1 = tap_value(c, ki - 1, kj - 1)
                v2 = tap_value(c, 2 * ki - 1, 2 * kj - 1)
                rows.append(
                    jnp.where(is0, v0_cache[c], jnp.where(is1, v1, v2)))
            t8 = jnp.stack(rows, axis=0).reshape(8, su * l_dim)
            o_ref[0, 8 * g:8 * g + 8, u0 * l_dim:(u0 + su) * l_dim] = t8


def _unfold_one_shard(xf, df, *, w, su):
    Bs, C, U, L = xf.shape
    F = U * L
    cb = 8
    return pl.pallas_call(
        functools.partial(_unfold_body, cb=cb, w=w, u_dim=U, l_dim=L, su=su),
        grid=(Bs, C // cb),
        in_specs=[
            pl.BlockSpec((1, cb, U, L), lambda b, c: (b, c, 0, 0)),
            pl.BlockSpec((1, U, L), lambda b, c: (b, 0, 0)),
        ],
        out_specs=pl.BlockSpec((1, cb * _K * _K, F), lambda b, c: (b, c, 0)),
        out_shape=jax.ShapeDtypeStruct((Bs, C * _K * _K, F), xf.dtype),
    )(xf, df)


def _sc_unfold(x2, d2, *, w):
    from jax import lax
    from jax.experimental.pallas import tpu as pltpu
    from jax.experimental.pallas import tpu_sc as plsc

    BC, F = x2.shape
    B = d2.shape[0]
    C = BC // B
    NW = 32
    PAIRS = BC // NW
    CF = 7168
    NCH = F // CF
    LEAD, XSZ = 256, 256 + F + 1024

    mesh = plsc.VectorSubcoreMesh(core_axis_name="c", subcore_axis_name="s")

    @functools.partial(
        pl.kernel,
        out_type=jax.ShapeDtypeStruct((B * C * 9, F), jnp.float32),
        mesh=mesh,
        scratch_types=[
            pltpu.VMEM((XSZ,), jnp.float32),
            pltpu.VMEM((CF,), jnp.int32),
            pltpu.VMEM((9 * CF,), jnp.float32),
            pltpu.SemaphoreType.DMA,
        ],
    )
    def k(x_hbm, d_hbm, o_hbm, xs, dsb, ob, sem):
        wid = lax.axis_index("s") * 2 + lax.axis_index("c")
        zero16 = jnp.zeros((16,), jnp.float32)

        @pl.loop(0, LEAD, step=16)
        def _(i):
            xs[pl.ds(i, 16)] = zero16

        @pl.loop(256 + F, XSZ, step=16)
        def _(i):
            xs[pl.ds(i, 16)] = zero16

        @pl.loop(0, PAIRS)
        def _(p):
            pr = wid * PAIRS + p
            b = pr // C
            c = pr - b * C
            pltpu.sync_copy(x_hbm.at[pr], xs.at[pl.ds(LEAD, F)])

            @pl.loop(0, NCH)
            def _(chi):
                f0 = chi * CF
                pltpu.sync_copy(d_hbm.at[b, pl.ds(f0, CF)], dsb)

                @pl.loop(0, CF, step=16)
                def _(i):
                    base = LEAD + f0 + i
                    d16 = dsb[pl.ds(i, 16)]
                    is0 = d16 == 0
                    is1 = d16 == 1
                    fv = lax.iota(jnp.int32, 16) + (f0 + i)
                    j16 = lax.rem(fv, w)
                    ok = {-1: j16 >= 1, 1: j16 <= w - 2, 3: j16 <= w - 4}
                    v0 = jnp.where(ok[-1], xs[pl.ds(base - w - 1, 16)], 0.0)
                    for t in range(9):
                        ki, kj = t // 3, t % 3
                        k1 = (ki - 1) * w + (kj - 1)
                        k2 = (2 * ki - 1) * w + (2 * kj - 1)
                        v1 = xs[pl.ds(base + k1, 16)]
                        if kj != 1:
                            v1 = jnp.where(ok[kj - 1], v1, 0.0)
                        v2 = jnp.where(ok[2 * kj - 1],
                                       xs[pl.ds(base + k2, 16)], 0.0)
                        ob[pl.ds(t * CF + i, 16)] = jnp.where(
                            is0, v0, jnp.where(is1, v1, v2))

                for t in range(9):
                    pltpu.sync_copy(
                        ob.at[pl.ds(t * CF, CF)], o_hbm.at[b * C * 9 + c * 9 + t,
                                           pl.ds(f0, CF)])

    return k(x2, d2)


@jax.jit
def kernel(input, dilation_map):
    B, C, H, W = input.shape
    F = H * W
    L = 128
    U = F // L

    out = _sc_unfold(input.reshape(B * C, F), dilation_map.reshape(B, F), w=W)
    return out.reshape(B, C * _K * _K, F)


# speedup vs baseline: 3.2309x; 3.2309x over previous
import functools

import jax
import jax.numpy as jnp
from jax.experimental import pallas as pl
_K = 3


def _unfold_body(x_ref, d_ref, o_ref, *, cb, w, u_dim, l_dim, su):
    nstrips = u_dim // su
    n_groups = cb * _K * _K // 8
    for st in range(nstrips):
        u0 = st * su
        ds = d_ref[0, u0:u0 + su, :]
        is0 = (ds == 0)
        is1 = (ds == 1)
        is2 = (ds == 2)
        fi = ((jax.lax.broadcasted_iota(jnp.int32, (su, l_dim), 0) + u0) * l_dim
              + jax.lax.broadcasted_iota(jnp.int32, (su, l_dim), 1))
        j = fi - (fi // w) * w
        col_ok = {s: (j + s >= 0) & (j + s < w) for s in (-1, 1, 3)}
        m0 = is0 & col_ok[-1]
        m1 = {-1: is1 & col_ok[-1], 0: is1, 1: is1 & col_ok[1]}
        m2 = {-1: is2 & col_ok[-1], 1: is2 & col_ok[1], 3: is2 & col_ok[3]}
        zero = jnp.zeros((su, l_dim), jnp.float32)

        xs_cache = {}

        def get_xs(c):
            if c not in xs_cache:
                lo, hi = max(u0 - 2, 0), min(u0 + su + 6, u_dim)
                v = x_ref[0, c, lo:hi, :]
                if u0 - 2 < 0:
                    v = jnp.concatenate(
                        [jnp.zeros((2 - u0, l_dim), v.dtype), v], axis=0)
                if u0 + su + 6 > u_dim:
                    v = jnp.concatenate(
                        [v, jnp.zeros((u0 + su + 6 - u_dim, l_dim), v.dtype)],
                        axis=0)
                xs_cache[c] = v
            return xs_cache[c]

        shift_cache = {}

        def tap_value(c, r, s):
            k = r * w + s
            if (c, k) in shift_cache:
                return shift_cache[(c, k)]
            xs = get_xs(c)
            q, m = divmod(k, l_dim)
            if m == 0:
                v = xs[2 + q:2 + q + su, :]
            else:
                w2 = jnp.concatenate(
                    [xs[2 + q:2 + q + su, :], xs[3 + q:3 + q + su, :]], axis=1)
                v = jax.lax.slice(w2, (0, m), (su, m + l_dim))
            shift_cache[(c, k)] = v
            return v

        for g in range(n_groups):
            rows = []
            for sub in range(8):
                rl = 8 * g + sub
                c, t = rl // 9, rl % 9
                ki, kj = t // _K, t % _K
                v0 = tap_value(c, -1, -1)
                v1 = tap_value(c, ki - 1, kj - 1)
                v2 = tap_value(c, 2 * ki - 1, 2 * kj - 1)
                rows.append(
                    jnp.where(m0, v0,
                              jnp.where(m1[kj - 1], v1,
                                        jnp.where(m2[2 * kj - 1], v2, zero))))
            t8 = jnp.stack(rows, axis=0).reshape(8, su * l_dim)
            o_ref[0, 8 * g:8 * g + 8, u0 * l_dim:(u0 + su) * l_dim] = t8


def _unfold_one_shard(xf, df, *, w, su):
    Bs, C, U, L = xf.shape
    F = U * L
    cb = 8
    return pl.pallas_call(
        functools.partial(_unfold_body, cb=cb, w=w, u_dim=U, l_dim=L, su=su),
        grid=(Bs, C // cb),
        in_specs=[
            pl.BlockSpec((1, cb, U, L), lambda b, c: (b, c, 0, 0)),
            pl.BlockSpec((1, U, L), lambda b, c: (b, 0, 0)),
        ],
        out_specs=pl.BlockSpec((1, cb * _K * _K, F), lambda b, c: (b, c, 0)),
        out_shape=jax.ShapeDtypeStruct((Bs, C * _K * _K, F), xf.dtype),
    )(xf, df)


@jax.jit
def kernel(input, dilation_map):
    B, C, H, W = input.shape
    F = H * W
    L = 128
    U = F // L

    fn = functools.partial(_unfold_one_shard, w=W, su=14)
    return fn(input.reshape(B, C, U, L), dilation_map.reshape(B, U, L))


# speedup vs baseline: 3.3328x; 1.0315x over previous
import functools

import jax
import jax.numpy as jnp
from jax.experimental import pallas as pl
_K = 3


def _unfold_body(x_ref, d_ref, o_ref, *, cb, w, u_dim, l_dim, su):
    nstrips = u_dim // su
    n_groups = cb * _K * _K // 8
    for st in range(nstrips):
        u0 = st * su
        ds = d_ref[0, u0:u0 + su, :]
        is0 = (ds == 0)
        is1 = (ds == 1)
        is2 = (ds == 2)
        fi = ((jax.lax.broadcasted_iota(jnp.int32, (su, l_dim), 0) + u0) * l_dim
              + jax.lax.broadcasted_iota(jnp.int32, (su, l_dim), 1))
        j = fi - (fi // w) * w
        col_ok = {s: (j + s >= 0) & (j + s < w) for s in (-1, 1, 3)}
        m0 = is0 & col_ok[-1]
        m1 = {-1: is1 & col_ok[-1], 0: is1, 1: is1 & col_ok[1]}
        m2 = {-1: is2 & col_ok[-1], 1: is2 & col_ok[1], 3: is2 & col_ok[3]}
        zero = jnp.zeros((su, l_dim), jnp.float32)

        xs_cache = {}

        def get_xs(c):
            if c not in xs_cache:
                lo, hi = max(u0 - 2, 0), min(u0 + su + 6, u_dim)
                v = x_ref[0, c, lo:hi, :]
                if u0 - 2 < 0:
                    v = jnp.concatenate(
                        [jnp.zeros((2 - u0, l_dim), v.dtype), v], axis=0)
                if u0 + su + 6 > u_dim:
                    v = jnp.concatenate(
                        [v, jnp.zeros((u0 + su + 6 - u_dim, l_dim), v.dtype)],
                        axis=0)
                xs_cache[c] = v
            return xs_cache[c]

        shift_cache = {}

        def tap_value(c, r, s):
            k = r * w + s
            if (c, k) in shift_cache:
                return shift_cache[(c, k)]
            xs = get_xs(c)
            q, m = divmod(k, l_dim)

            def sh(qq, mm):
                v = xs[2 + qq:2 + qq + su,
                       max(mm, 0):l_dim + min(mm, 0)]
                if mm > 0:
                    v = jnp.concatenate(
                        [v, jnp.zeros((su, mm), v.dtype)], axis=1)
                elif mm < 0:
                    v = jnp.concatenate(
                        [jnp.zeros((su, -mm), v.dtype), v], axis=1)
                return v

            v = sh(q, m) if m == 0 else sh(q, m) + sh(q + 1, m - l_dim)
            shift_cache[(c, k)] = v
            return v

        for g in range(n_groups):
            rows = []
            for sub in range(8):
                rl = 8 * g + sub
                c, t = rl // 9, rl % 9
                ki, kj = t // _K, t % _K
                v0 = tap_value(c, -1, -1)
                v1 = tap_value(c, ki - 1, kj - 1)
                v2 = tap_value(c, 2 * ki - 1, 2 * kj - 1)
                rows.append(
                    jnp.where(m0, v0,
                              jnp.where(m1[kj - 1], v1,
                                        jnp.where(m2[2 * kj - 1], v2, zero))))
            t8 = jnp.stack(rows, axis=0).reshape(8, su * l_dim)
            o_ref[0, 8 * g:8 * g + 8, u0 * l_dim:(u0 + su) * l_dim] = t8


def _unfold_one_shard(xf, df, *, w, su):
    Bs, C, U, L = xf.shape
    F = U * L
    cb = 8
    return pl.pallas_call(
        functools.partial(_unfold_body, cb=cb, w=w, u_dim=U, l_dim=L, su=su),
        grid=(Bs, C // cb),
        in_specs=[
            pl.BlockSpec((1, cb, U, L), lambda b, c: (b, c, 0, 0)),
            pl.BlockSpec((1, U, L), lambda b, c: (b, 0, 0)),
        ],
        out_specs=pl.BlockSpec((1, cb * _K * _K, F), lambda b, c: (b, c, 0)),
        out_shape=jax.ShapeDtypeStruct((Bs, C * _K * _K, F), xf.dtype),
    )(xf, df)


@jax.jit
def kernel(input, dilation_map):
    B, C, H, W = input.shape
    F = H * W
    L = 128
    U = F // L

    fn = functools.partial(_unfold_one_shard, w=W, su=14)
    return fn(input.reshape(B, C, U, L), dilation_map.reshape(B, U, L))


# speedup vs baseline: 3.4107x; 1.0234x over previous
import functools

import jax
import jax.numpy as jnp
from jax.experimental import pallas as pl
_K = 3


def _unfold_body(x_ref, d_ref, o_ref, *, cb, w, u_dim, l_dim, su):
    nstrips = u_dim // su
    n_groups = cb * _K * _K // 8
    for st in range(nstrips):
        u0 = st * su
        ds = d_ref[0, u0:u0 + su, :]
        is0 = (ds == 0)
        is1 = (ds == 1)
        fi = ((jax.lax.broadcasted_iota(jnp.int32, (su, l_dim), 0) + u0) * l_dim
              + jax.lax.broadcasted_iota(jnp.int32, (su, l_dim), 1))
        j = fi - (fi // w) * w
        col_ok = {s: (j + s >= 0) & (j + s < w) for s in (-1, 1, 3)}

        xs_cache = {}

        def get_xs(c):
            if c not in xs_cache:
                lo, hi = max(u0 - 2, 0), min(u0 + su + 6, u_dim)
                v = x_ref[0, c, lo:hi, :]
                if u0 - 2 < 0:
                    v = jnp.concatenate(
                        [jnp.zeros((2 - u0, l_dim), v.dtype), v], axis=0)
                if u0 + su + 6 > u_dim:
                    v = jnp.concatenate(
                        [v, jnp.zeros((u0 + su + 6 - u_dim, l_dim), v.dtype)],
                        axis=0)
                xs_cache[c] = v
            return xs_cache[c]

        v0_cache = {}

        def tap_value(c, r, s):
            xs = get_xs(c)
            q, m = divmod(r * w + s, l_dim)

            def sh(qq, mm):
                v = xs[2 + qq:2 + qq + su,
                       max(mm, 0):l_dim + min(mm, 0)]
                if mm > 0:
                    v = jnp.concatenate(
                        [v, jnp.zeros((su, mm), v.dtype)], axis=1)
                elif mm < 0:
                    v = jnp.concatenate(
                        [jnp.zeros((su, -mm), v.dtype), v], axis=1)
                return v

            v = sh(q, m) if m == 0 else sh(q, m) + sh(q + 1, m - l_dim)
            if s in col_ok:
                v = jnp.where(col_ok[s], v, 0.0)
            return v

        for g in range(n_groups):
            rows = []
            for sub in range(8):
                rl = 8 * g + sub
                c, t = rl // 9, rl % 9
                ki, kj = t // _K, t % _K
                if c not in v0_cache:
                    v0_cache[c] = tap_value(c, -1, -1)
                v1 = tap_value(c, ki - 1, kj - 1)
                v2 = tap_value(c, 2 * ki - 1, 2 * kj - 1)
                rows.append(
                    jnp.where(is0, v0_cache[c], jnp.where(is1, v1, v2)))
            t8 = jnp.stack(rows, axis=0).reshape(8, su * l_dim)
            o_ref[0, 8 * g:8 * g + 8, u0 * l_dim:(u0 + su) * l_dim] = t8


def _unfold_one_shard(xf, df, *, w, su):
    Bs, C, U, L = xf.shape
    F = U * L
    cb = 8
    return pl.pallas_call(
        functools.partial(_unfold_body, cb=cb, w=w, u_dim=U, l_dim=L, su=su),
        grid=(Bs, C // cb),
        in_specs=[
            pl.BlockSpec((1, cb, U, L), lambda b, c: (b, c, 0, 0)),
            pl.BlockSpec((1, U, L), lambda b, c: (b, 0, 0)),
        ],
        out_specs=pl.BlockSpec((1, cb * _K * _K, F), lambda b, c: (b, c, 0)),
        out_shape=jax.ShapeDtypeStruct((Bs, C * _K * _K, F), xf.dtype),
    )(xf, df)


@jax.jit
def kernel(input, dilation_map):
    B, C, H, W = input.shape
    F = H * W
    L = 128
    U = F // L

    fn = functools.partial(_unfold_one_shard, w=W, su=14)
    return fn(input.reshape(B, C, U, L), dilation_map.reshape(B, U, L))
